# baseline (speedup 1.0000x reference)
import functools

import jax
import jax.numpy as jnp
from jax.experimental import pallas as pl
from jax.experimental.pallas import tpu as pltpu

NUM_EXPERTS = 64
TOP_K = 8
N_GROUP = 8
TOPK_GROUP = 4
EPG = NUM_EXPERTS // N_GROUP
ROUTED_SCALING = 2.5

TILE_T = 2048


def _router_body(h_ref, w_ref, b_ref, brow_ref, rw_ref, idx_ref):
    t = h_ref.shape[0]
    logits = jax.lax.dot_general(
        w_ref[...], h_ref[...],
        dimension_numbers=(((1,), (1,)), ((), ())),
        preferred_element_type=jnp.float32,
    )
    s = jax.nn.sigmoid(logits)
    sfc = s + b_ref[...]

    NEG = jnp.int32(-2147483647 - 1)
    qsfc = ((sfc - 0.5) * 33554432.0).astype(jnp.int32).reshape(
        EPG, N_GROUP, t)
    jj = jax.lax.broadcasted_iota(jnp.int32, (EPG, N_GROUP, t), 0)
    gg = jax.lax.broadcasted_iota(jnp.int32, (EPG, N_GROUP, t), 1)
    eo3 = EPG * gg + jj
    k3 = (qsfc << 6) | (63 - eo3)

    sfc3 = sfc.reshape(EPG, N_GROUP, t)
    m1f = sfc3[0]
    m2f = jnp.full((N_GROUP, t), -jnp.inf, dtype=jnp.float32)
    for j in range(1, EPG):
        a = sfc3[j]
        m2f = jnp.maximum(m2f, jnp.minimum(m1f, a))
        m1f = jnp.maximum(m1f, a)
    gs = m1f + m2f
    gi = jax.lax.broadcasted_iota(jnp.int32, (N_GROUP, t), 0)
    gkey = (((gs - 1.0) * 134217728.0).astype(jnp.int32) << 3) | (7 - gi)

    gkey_p = gkey.reshape(N_GROUP, 1, t)
    beats = (gkey_p > gkey[None, :, :]).astype(jnp.int32)
    rank = jnp.sum(beats, axis=0)
    gmask = rank < TOPK_GROUP

    ZKEY = jnp.int32(-16777216 << 6)
    key = jnp.where(gmask[None, :, :], k3, ZKEY | (63 - eo3))

    idx_rows = []
    rw_rows = []
    for _ in range(TOP_K):
        kmax = jnp.max(jnp.max(key, axis=0), axis=0)
        sel = 63 - (kmax & 63)
        hit = key == kmax[None, None, :]
        sfc_sel = (kmax >> 6).astype(jnp.float32) * (1.0 / 33554432.0) + 0.5
        hitf = hit.reshape(NUM_EXPERTS, t).astype(jnp.float32)
        b_sel = jax.lax.dot_general(
            brow_ref[...], hitf,
            dimension_numbers=(((1,), (0,)), ((), ())),
            preferred_element_type=jnp.float32,
        )[0]
        rw_rows.append(sfc_sel - b_sel)
        key = jnp.where(hit, NEG, key)
        idx_rows.append(sel)

    rws = jnp.stack(rw_rows, axis=0)
    denom = jnp.sum(rws, axis=0) + 1e-20
    rw_ref[...] = rws * (ROUTED_SCALING / denom)[None, :]
    idx_ref[...] = jnp.stack(idx_rows, axis=0)


@functools.partial(jax.jit, static_argnames=())
def kernel(hidden_states, weight, e_score_correction_bias):
    T, H = hidden_states.shape
    E = weight.shape[0]
    n_tiles = T // TILE_T
    w_perm = weight.reshape(N_GROUP, EPG, H).transpose(1, 0, 2).reshape(E, H)
    b_perm = e_score_correction_bias.reshape(N_GROUP, EPG).T.reshape(E, 1)

    rw_t, idx_t = pl.pallas_call(
        _router_body,
        grid=(n_tiles,),
        in_specs=[
            pl.BlockSpec((TILE_T, H), lambda i: (i, 0)),
            pl.BlockSpec((E, H), lambda i: (0, 0)),
            pl.BlockSpec((E, 1), lambda i: (0, 0)),
            pl.BlockSpec((1, E), lambda i: (0, 0)),
        ],
        out_specs=[
            pl.BlockSpec((TOP_K, TILE_T), lambda i: (0, i)),
            pl.BlockSpec((TOP_K, TILE_T), lambda i: (0, i)),
        ],
        out_shape=[
            jax.ShapeDtypeStruct((TOP_K, T), jnp.float32),
            jax.ShapeDtypeStruct((TOP_K, T), jnp.int32),
        ],
    )(hidden_states, w_perm, b_perm, b_perm.reshape(1, E))

    return rw_t.T, idx_t.T


# speedup vs baseline: 1.1026x; 1.1026x over previous
import functools

import jax
import jax.numpy as jnp
from jax.experimental import pallas as pl
from jax.experimental.pallas import tpu as pltpu

NUM_EXPERTS = 64
TOP_K = 8
N_GROUP = 8
TOPK_GROUP = 4
EPG = NUM_EXPERTS // N_GROUP
ROUTED_SCALING = 2.5

TILE_T = 4096


def _router_body(h_ref, w_ref, b_ref, brow_ref, rw_ref, idx_ref):
    t = h_ref.shape[0]
    logits = jax.lax.dot_general(
        w_ref[...], h_ref[...],
        dimension_numbers=(((1,), (1,)), ((), ())),
        preferred_element_type=jnp.float32,
    )
    s = jax.nn.sigmoid(logits)
    sfc = s + b_ref[...]

    NEG = jnp.int32(-2147483647 - 1)
    qsfc = ((sfc - 0.5) * 33554432.0).astype(jnp.int32).reshape(
        EPG, N_GROUP, t)
    jj = jax.lax.broadcasted_iota(jnp.int32, (EPG, N_GROUP, t), 0)
    gg = jax.lax.broadcasted_iota(jnp.int32, (EPG, N_GROUP, t), 1)
    eo3 = EPG * gg + jj
    k3 = (qsfc << 6) | (63 - eo3)

    sfc3 = sfc.reshape(EPG, N_GROUP, t)
    m1f = sfc3[0]
    m2f = jnp.full((N_GROUP, t), -jnp.inf, dtype=jnp.float32)
    for j in range(1, EPG):
        a = sfc3[j]
        m2f = jnp.maximum(m2f, jnp.minimum(m1f, a))
        m1f = jnp.maximum(m1f, a)
    gs = m1f + m2f
    gi = jax.lax.broadcasted_iota(jnp.int32, (N_GROUP, t), 0)
    gkey = (((gs - 1.0) * 134217728.0).astype(jnp.int32) << 3) | (7 - gi)

    gkey_p = gkey.reshape(N_GROUP, 1, t)
    beats = (gkey_p > gkey[None, :, :]).astype(jnp.int32)
    rank = jnp.sum(beats, axis=0)
    gmask = rank < TOPK_GROUP

    ZKEY = jnp.int32(-16777216 << 6)
    key = jnp.where(gmask[None, :, :], k3, ZKEY | (63 - eo3))

    idx_rows = []
    rw_rows = []
    for _ in range(TOP_K):
        kmax = jnp.max(jnp.max(key, axis=0), axis=0)
        sel = 63 - (kmax & 63)
        hit = key == kmax[None, None, :]
        sfc_sel = (kmax >> 6).astype(jnp.float32) * (1.0 / 33554432.0) + 0.5
        hitf = hit.reshape(NUM_EXPERTS, t).astype(jnp.float32)
        b_sel = jax.lax.dot_general(
            brow_ref[...], hitf,
            dimension_numbers=(((1,), (0,)), ((), ())),
            preferred_element_type=jnp.float32,
        )[0]
        rw_rows.append(sfc_sel - b_sel)
        key = jnp.where(hit, NEG, key)
        idx_rows.append(sel)

    rws = jnp.stack(rw_rows, axis=0)
    denom = jnp.sum(rws, axis=0) + 1e-20
    rw_ref[...] = rws * (ROUTED_SCALING / denom)[None, :]
    idx_ref[...] = jnp.stack(idx_rows, axis=0)


@functools.partial(jax.jit, static_argnames=())
def kernel(hidden_states, weight, e_score_correction_bias):
    T, H = hidden_states.shape
    E = weight.shape[0]
    n_tiles = T // TILE_T
    w_perm = weight.reshape(N_GROUP, EPG, H).transpose(1, 0, 2).reshape(E, H)
    b_perm = e_score_correction_bias.reshape(N_GROUP, EPG).T.reshape(E, 1)

    rw_t, idx_t = pl.pallas_call(
        _router_body,
        grid=(n_tiles,),
        in_specs=[
            pl.BlockSpec((TILE_T, H), lambda i: (i, 0)),
            pl.BlockSpec((E, H), lambda i: (0, 0)),
            pl.BlockSpec((E, 1), lambda i: (0, 0)),
            pl.BlockSpec((1, E), lambda i: (0, 0)),
        ],
        out_specs=[
            pl.BlockSpec((TOP_K, TILE_T), lambda i: (0, i)),
            pl.BlockSpec((TOP_K, TILE_T), lambda i: (0, i)),
        ],
        out_shape=[
            jax.ShapeDtypeStruct((TOP_K, T), jnp.float32),
            jax.ShapeDtypeStruct((TOP_K, T), jnp.int32),
        ],
    )(hidden_states, w_perm, b_perm, b_perm.reshape(1, E))

    return rw_t.T, idx_t.T
